# Initial kernel scaffold; baseline (speedup 1.0000x reference)
import jax
import jax.numpy as jnp
from jax.experimental import pallas as pl


def kernel(indices, table):
    raise NotImplementedError("write your pallas kernel here")


# speedup vs baseline: 1.7963x; 1.7963x over previous
import functools

import jax
import jax.numpy as jnp
from jax import lax
from jax.experimental import pallas as pl
from jax.experimental.pallas import tpu as pltpu
from jax.experimental.pallas import tpu_sc as plsc

NUM_WORKERS = 32
CHUNK = 512


@functools.partial(jax.jit, static_argnums=(2, 3))
def _gather_rows(idx_flat, table, n_rows, dim):
    per_w = n_rows // NUM_WORKERS
    n_chunks = per_w // CHUNK
    mesh = plsc.VectorSubcoreMesh(core_axis_name="c", subcore_axis_name="s")

    @functools.partial(
        pl.kernel,
        mesh=mesh,
        out_type=jax.ShapeDtypeStruct((n_rows, dim), jnp.float32),
        scratch_types=[
            pltpu.VMEM((CHUNK,), jnp.int32),
            pltpu.VMEM((CHUNK, dim), jnp.float32),
            pltpu.SemaphoreType.DMA,
        ],
        compiler_params=pltpu.CompilerParams(use_tc_tiling_on_sc=False),
    )
    def k(idx_hbm, table_hbm, out_hbm, idx_v, rows_v, sem):
        wid = lax.axis_index("s") * 2 + lax.axis_index("c")
        base = wid * per_w

        def body(g, carry):
            off = base + g * CHUNK
            pltpu.sync_copy(idx_hbm.at[pl.ds(off, CHUNK)], idx_v)
            pltpu.async_copy(table_hbm.at[idx_v], rows_v, sem).wait()
            pltpu.sync_copy(rows_v, out_hbm.at[pl.ds(off, CHUNK)])
            return carry

        lax.fori_loop(0, n_chunks, body, 0)

    return k(idx_flat, table)


def kernel(indices, table):
    b, h = indices.shape
    v, d = table.shape
    n = b * h
    out = _gather_rows(indices.reshape(n), table, n, d)
    return out.reshape(b, h, d)


# speedup vs baseline: 1.8757x; 1.0442x over previous
import functools

import jax
import jax.numpy as jnp
from jax import lax
from jax.experimental import pallas as pl
from jax.experimental.pallas import tpu as pltpu
from jax.experimental.pallas import tpu_sc as plsc

NUM_WORKERS = 32
CHUNK = 400
NBUF = 4


@functools.partial(jax.jit, static_argnums=(2, 3))
def _gather_rows(idx_flat, table, n_rows, dim):
    per_w = n_rows // NUM_WORKERS
    n_chunks = per_w // CHUNK
    n_groups = n_chunks // NBUF
    mesh = plsc.VectorSubcoreMesh(core_axis_name="c", subcore_axis_name="s")

    @functools.partial(
        pl.kernel,
        mesh=mesh,
        out_type=jax.ShapeDtypeStruct((n_rows, dim), jnp.float32),
        scratch_types=[
            pltpu.VMEM((per_w,), jnp.int32),
            [pltpu.VMEM((CHUNK, dim), jnp.float32) for _ in range(NBUF)],
            [pltpu.SemaphoreType.DMA for _ in range(NBUF)],
            [pltpu.SemaphoreType.DMA for _ in range(NBUF)],
        ],
        compiler_params=pltpu.CompilerParams(use_tc_tiling_on_sc=False),
    )
    def k(idx_hbm, table_hbm, out_hbm, idx_v, rows, semg, semw):
        wid = lax.axis_index("s") * 2 + lax.axis_index("c")
        base = wid * per_w

        def start_gather(c, p):
            return pltpu.async_copy(
                table_hbm.at[idx_v.at[pl.ds(c * CHUNK, CHUNK)]], rows[p], semg[p]
            )

        def wait_gather(c, p):
            pltpu.make_async_copy(
                table_hbm.at[idx_v.at[pl.ds(c * CHUNK, CHUNK)]], rows[p], semg[p]
            ).wait()

        def start_write(c, p):
            return pltpu.async_copy(
                rows[p], out_hbm.at[pl.ds(base + c * CHUNK, CHUNK)], semw[p]
            )

        def wait_write(c, p):
            pltpu.make_async_copy(
                rows[p], out_hbm.at[pl.ds(base + c * CHUNK, CHUNK)], semw[p]
            ).wait()

        pltpu.sync_copy(idx_hbm.at[pl.ds(base, per_w)], idx_v)

        for p in range(NBUF):
            start_gather(p, p)
            if p >= 2:
                wait_gather(p - 2, p - 2)
                start_write(p - 2, p - 2)

        def body(g, carry):
            for p in range(NBUF):
                c = g * NBUF + p
                wait_write(c - NBUF, p)
                start_gather(c, p)
                wait_gather(c - 2, (p - 2) % NBUF)
                start_write(c - 2, (p - 2) % NBUF)
            return carry

        lax.fori_loop(1, n_groups, body, 0, unroll=False)

        n = n_chunks
        for c in (n - 2, n - 1):
            p = c % NBUF
            wait_gather(c, p)
            start_write(c, p)
        for p in range(NBUF):
            wait_write(n - NBUF + p, p)

    return k(idx_flat, table)


def kernel(indices, table):
    b, h = indices.shape
    v, d = table.shape
    n = b * h
    out = _gather_rows(indices.reshape(n), table, n, d)
    return out.reshape(b, h, d)
